# baseline (speedup 1.0000x reference)
import functools

import jax
import jax.numpy as jnp
from jax import lax
from jax.experimental import pallas as pl
from jax.experimental.pallas import tpu as pltpu
from jax.experimental.pallas import tpu_sc as plsc

B = 64
D = 1024
HID = 1024
OUT = 1024
E = 64
K = 2

_LANE = 16
_NSUB = 16
_ROWS = B // _NSUB


def _gate_body(x_ref, wg_ref, bg_ref, h_out_ref, g_ref, idx_ref):
    h = jnp.dot(x_ref[...], wg_ref[...],
                preferred_element_type=jnp.float32) + bg_ref[...]
    h_out_ref[...] = h
    m1 = jnp.max(h, axis=1, keepdims=True)
    e_all = jnp.exp(h - m1)
    is_max = h == m1
    cnt = jnp.sum(is_max.astype(jnp.float32), axis=1, keepdims=True)
    m2 = jnp.max(jnp.where(is_max, -jnp.inf, h), axis=1, keepdims=True)
    kth = jnp.where(cnt >= 2.0, m1, m2)
    mask = h >= kth
    gnum = jnp.where(mask, e_all, 0.0)
    g_ref[...] = gnum / jnp.sum(gnum, axis=1, keepdims=True)

    active = jnp.max(mask.astype(jnp.float32), axis=0, keepdims=True)
    lt = (lax.broadcasted_iota(jnp.int32, (E, E), 0)
          <= lax.broadcasted_iota(jnp.int32, (E, E), 1)).astype(jnp.float32)
    c_row = jnp.dot(active, lt)
    count = jnp.sum(active)
    iota_row = lax.broadcasted_iota(jnp.int32, (1, E), 1).astype(jnp.float32)
    last = jnp.max(jnp.where(active > 0.0, iota_row, -1.0))
    j_sub = lax.broadcasted_iota(jnp.int32, (E, E), 0).astype(jnp.float32)
    e_lane = lax.broadcasted_iota(jnp.int32, (E, E), 1).astype(jnp.float32)
    slot = (c_row - 1.0 == j_sub) & (active > 0.0)
    idx_col = jnp.sum(jnp.where(slot, e_lane, 0.0), axis=1, keepdims=True)
    j_col = lax.broadcasted_iota(jnp.int32, (E, 1), 0).astype(jnp.float32)
    idx_ref[...] = jnp.where(j_col < count, idx_col, last).astype(jnp.int32)


def _sc_probs_body(h_hbm, probs_hbm, h_v, p_v):
    cid = lax.axis_index("c")
    sid = lax.axis_index("s")

    @pl.when(cid == 0)
    def _softmax():
        base = sid * _ROWS
        pltpu.sync_copy(h_hbm.at[pl.ds(base, _ROWS)], h_v)
        for r in range(_ROWS):
            hk = [h_v[r, pl.ds(k * _LANE, _LANE)] for k in range(E // _LANE)]
            m1 = functools.reduce(jnp.maximum, [jnp.max(h) for h in hk])
            ek = [jnp.exp(h - m1) for h in hk]
            s_all = sum(jnp.sum(e) for e in ek)
            for k in range(E // _LANE):
                p_v[r, pl.ds(k * _LANE, _LANE)] = ek[k] / s_all
        pltpu.sync_copy(p_v, probs_hbm.at[pl.ds(base, _ROWS)])


_S = 4
_DS = D // _S
_HS = HID // _S


def _expert_body(idx_ref, x_ref, g_ref, *rest):
    w1_refs = rest[0:_S]
    b1_ref = rest[_S]
    w2_refs = rest[_S + 1:2 * _S + 1]
    b2_ref = rest[2 * _S + 1]
    wf_ref = rest[2 * _S + 2]
    bf_ref = rest[2 * _S + 3]
    out_ref = rest[2 * _S + 4]
    acc_ref = rest[2 * _S + 5]
    xb_ref = rest[2 * _S + 6]
    h1b_ref = rest[2 * _S + 7]

    i = pl.program_id(0)
    e = idx_ref[i]
    prev = idx_ref[jnp.maximum(i - 1, 0)]
    is_new = (i == 0) | (e != prev)

    @pl.when(i == 0)
    def _init():
        acc_ref[...] = jnp.zeros_like(acc_ref)
        xb_ref[...] = x_ref[...].astype(jnp.bfloat16)

    @pl.when(is_new)
    def _compute():
        h1 = b1_ref[0, 0] + jnp.zeros((B, HID), jnp.float32)
        for j in range(_S):
            xj = xb_ref[:, pl.ds(j * _DS, _DS)]
            h1 += jnp.dot(xj, w1_refs[j][0, 0].astype(jnp.bfloat16),
                          preferred_element_type=jnp.float32)
        h1b_ref[...] = jnp.maximum(h1, 0.0).astype(jnp.bfloat16)
        eo = b2_ref[0, 0] + jnp.zeros((B, HID), jnp.float32)
        for j in range(_S):
            hj = h1b_ref[:, pl.ds(j * _HS, _HS)]
            eo += jnp.dot(hj, w2_refs[j][0, 0].astype(jnp.bfloat16),
                          preferred_element_type=jnp.float32)
        lane = lax.broadcasted_iota(jnp.int32, (B, E), 1)
        gate = jnp.sum(jnp.where(lane == e, g_ref[...], 0.0), axis=1,
                       keepdims=True)
        acc_ref[...] += gate * eo

    @pl.when(i == E - 1)
    def _final():
        out_ref[...] = jnp.dot(acc_ref[...], wf_ref[...],
                               preferred_element_type=jnp.float32) + bf_ref[...]


def kernel(x_list, Wg, bg, W1, b1, W2, b2, Wf, bf):
    x = x_list.reshape(B, D)

    H, G, idx2d = pl.pallas_call(
        _gate_body,
        out_shape=(
            jax.ShapeDtypeStruct((B, E), jnp.float32),
            jax.ShapeDtypeStruct((B, E), jnp.float32),
            jax.ShapeDtypeStruct((E, 1), jnp.int32),
        ),
    )(x, Wg, bg.reshape(1, E))
    idx = idx2d.reshape(E)

    sc_probs = pl.kernel(
        _sc_probs_body,
        out_type=jax.ShapeDtypeStruct((B, E), jnp.float32),
        mesh=plsc.VectorSubcoreMesh(core_axis_name="c", subcore_axis_name="s"),
        compiler_params=pltpu.CompilerParams(needs_layout_passes=False),
        scratch_types=[
            pltpu.VMEM((_ROWS, E), jnp.float32),
            pltpu.VMEM((_ROWS, E), jnp.float32),
        ],
    )
    probs = sc_probs(H)

    def _wspec(j):
        return pl.BlockSpec((1, 1, _DS, HID),
                            lambda i, idx_ref, j=j: (idx_ref[i], j, 0, 0))

    grid_spec = pltpu.PrefetchScalarGridSpec(
        num_scalar_prefetch=1,
        grid=(E,),
        in_specs=(
            [pl.BlockSpec((B, D), lambda i, idx_ref: (0, 0)),
             pl.BlockSpec((B, E), lambda i, idx_ref: (0, 0))]
            + [_wspec(j) for j in range(_S)]
            + [pl.BlockSpec((1, 1, HID), lambda i, idx_ref: (idx_ref[i], 0, 0))]
            + [_wspec(j) for j in range(_S)]
            + [pl.BlockSpec((1, 1, HID), lambda i, idx_ref: (idx_ref[i], 0, 0)),
               pl.BlockSpec((HID, OUT), lambda i, idx_ref: (0, 0)),
               pl.BlockSpec((1, OUT), lambda i, idx_ref: (0, 0))]
        ),
        out_specs=pl.BlockSpec((B, OUT), lambda i, idx_ref: (0, 0)),
        scratch_shapes=[
            pltpu.VMEM((B, HID), jnp.float32),
            pltpu.VMEM((B, D), jnp.bfloat16),
            pltpu.VMEM((B, HID), jnp.bfloat16),
        ],
    )
    w1r = W1.reshape(E, _S, _DS, HID)
    w2r = W2.reshape(E, _S, _HS, HID)
    out = pl.pallas_call(
        _expert_body,
        grid_spec=grid_spec,
        out_shape=jax.ShapeDtypeStruct((B, OUT), jnp.float32),
    )(idx, x, G,
      *[w1r] * _S, b1.reshape(E, 1, HID),
      *[w2r] * _S, b2.reshape(E, 1, HID),
      Wf, bf.reshape(1, OUT))

    return (out, probs.reshape(1, B, E))


# speedup vs baseline: 1.0665x; 1.0665x over previous
import functools

import jax
import jax.numpy as jnp
from jax import lax
from jax.experimental import pallas as pl
from jax.experimental.pallas import tpu as pltpu
from jax.experimental.pallas import tpu_sc as plsc

B = 64
D = 1024
HID = 1024
OUT = 1024
E = 64
K = 2

_LANE = 16
_NSUB = 16
_ROWS = B // _NSUB


def _gate_body(x_ref, wg_ref, bg_ref, h_out_ref, g_ref, idx_ref):
    h = jnp.dot(x_ref[...], wg_ref[...],
                preferred_element_type=jnp.float32) + bg_ref[...]
    h_out_ref[...] = h
    m1 = jnp.max(h, axis=1, keepdims=True)
    e_all = jnp.exp(h - m1)
    is_max = h == m1
    cnt = jnp.sum(is_max.astype(jnp.float32), axis=1, keepdims=True)
    m2 = jnp.max(jnp.where(is_max, -jnp.inf, h), axis=1, keepdims=True)
    kth = jnp.where(cnt >= 2.0, m1, m2)
    mask = h >= kth
    gnum = jnp.where(mask, e_all, 0.0)
    g_ref[...] = gnum / jnp.sum(gnum, axis=1, keepdims=True)

    active = jnp.max(mask.astype(jnp.float32), axis=0, keepdims=True)
    lt = (lax.broadcasted_iota(jnp.int32, (E, E), 0)
          <= lax.broadcasted_iota(jnp.int32, (E, E), 1)).astype(jnp.float32)
    c_row = jnp.dot(active, lt)
    count = jnp.sum(active)
    iota_row = lax.broadcasted_iota(jnp.int32, (1, E), 1).astype(jnp.float32)
    last = jnp.max(jnp.where(active > 0.0, iota_row, -1.0))
    j_sub = lax.broadcasted_iota(jnp.int32, (E, E), 0).astype(jnp.float32)
    e_lane = lax.broadcasted_iota(jnp.int32, (E, E), 1).astype(jnp.float32)
    slot = (c_row - 1.0 == j_sub) & (active > 0.0)
    idx_col = jnp.sum(jnp.where(slot, e_lane, 0.0), axis=1, keepdims=True)
    j_col = lax.broadcasted_iota(jnp.int32, (E, 1), 0).astype(jnp.float32)
    idx_ref[...] = jnp.where(j_col < count, idx_col, last).astype(jnp.int32)


def _sc_probs_body(h_hbm, probs_hbm, h_v, p_v):
    cid = lax.axis_index("c")
    sid = lax.axis_index("s")

    @pl.when(cid == 0)
    def _softmax():
        base = sid * _ROWS
        pltpu.sync_copy(h_hbm.at[pl.ds(base, _ROWS)], h_v)
        for r in range(_ROWS):
            hk = [h_v[r, pl.ds(k * _LANE, _LANE)] for k in range(E // _LANE)]
            m1 = functools.reduce(jnp.maximum, [jnp.max(h) for h in hk])
            ek = [jnp.exp(h - m1) for h in hk]
            s_all = sum(jnp.sum(e) for e in ek)
            for k in range(E // _LANE):
                p_v[r, pl.ds(k * _LANE, _LANE)] = ek[k] / s_all
        pltpu.sync_copy(p_v, probs_hbm.at[pl.ds(base, _ROWS)])


def _expert_body(idx_ref, x_ref, g_ref, w1_ref, b1_ref, w2_ref, b2_ref,
                 wf_ref, bf_ref, out_ref, acc_ref, xb_ref):
    i = pl.program_id(0)
    e = idx_ref[i, 0]
    prev = idx_ref[jnp.maximum(i - 1, 0), 0]
    is_new = (i == 0) | (e != prev)

    @pl.when(i == 0)
    def _init():
        acc_ref[...] = jnp.zeros_like(acc_ref)
        xb_ref[...] = x_ref[...].astype(jnp.bfloat16)

    @pl.when(is_new)
    def _compute():
        w1 = w1_ref[0].astype(jnp.bfloat16)
        h1 = jnp.dot(xb_ref[...], w1, preferred_element_type=jnp.float32)
        h1 = jnp.maximum(h1 + b1_ref[pl.ds(e, 1), :], 0.0)
        w2 = w2_ref[0].astype(jnp.bfloat16)
        eo = (jnp.dot(h1.astype(jnp.bfloat16), w2,
                      preferred_element_type=jnp.float32)
              + b2_ref[pl.ds(e, 1), :])
        lane = lax.broadcasted_iota(jnp.int32, (B, E), 1)
        gate = jnp.sum(jnp.where(lane == e, g_ref[...], 0.0), axis=1,
                       keepdims=True)
        acc_ref[...] += gate * eo

    @pl.when(i == E - 1)
    def _final():
        out_ref[...] = (jnp.dot(acc_ref[...], wf_ref[...],
                                preferred_element_type=jnp.float32)
                        + bf_ref[...])


def kernel(x_list, Wg, bg, W1, b1, W2, b2, Wf, bf):
    x = x_list.reshape(B, D)

    H, G, idx = pl.pallas_call(
        _gate_body,
        out_shape=(
            jax.ShapeDtypeStruct((B, E), jnp.float32),
            jax.ShapeDtypeStruct((B, E), jnp.float32),
            jax.ShapeDtypeStruct((E, 1), jnp.int32),
        ),
    )(x, Wg, bg)

    sc_probs = pl.kernel(
        _sc_probs_body,
        out_type=jax.ShapeDtypeStruct((B, E), jnp.float32),
        mesh=plsc.VectorSubcoreMesh(core_axis_name="c", subcore_axis_name="s",
                                    num_cores=1),
        compiler_params=pltpu.CompilerParams(needs_layout_passes=False),
        scratch_types=[
            pltpu.VMEM((_ROWS, E), jnp.float32),
            pltpu.VMEM((_ROWS, E), jnp.float32),
        ],
    )
    probs = sc_probs(H)

    grid_spec = pltpu.PrefetchScalarGridSpec(
        num_scalar_prefetch=1,
        grid=(E,),
        in_specs=[
            pl.BlockSpec((B, D), lambda i, idx_ref: (0, 0)),
            pl.BlockSpec((B, E), lambda i, idx_ref: (0, 0)),
            pl.BlockSpec((1, D, HID), lambda i, idx_ref: (idx_ref[i, 0], 0, 0)),
            pl.BlockSpec((E, HID), lambda i, idx_ref: (0, 0)),
            pl.BlockSpec((1, HID, HID),
                         lambda i, idx_ref: (idx_ref[i, 0], 0, 0)),
            pl.BlockSpec((E, HID), lambda i, idx_ref: (0, 0)),
            pl.BlockSpec((HID, OUT), lambda i, idx_ref: (0, 0)),
            pl.BlockSpec((OUT,), lambda i, idx_ref: (0,)),
        ],
        out_specs=pl.BlockSpec((B, OUT), lambda i, idx_ref: (0, 0)),
        scratch_shapes=[
            pltpu.VMEM((B, HID), jnp.float32),
            pltpu.VMEM((B, D), jnp.bfloat16),
        ],
    )
    out = pl.pallas_call(
        _expert_body,
        grid_spec=grid_spec,
        out_shape=jax.ShapeDtypeStruct((B, OUT), jnp.float32),
    )(idx, x, G, W1, b1, W2, b2, Wf, bf)

    return (out, probs.reshape(1, B, E))
